# baseline (speedup 1.0000x reference)
import functools

import jax
import jax.numpy as jnp
from jax import lax
from jax.experimental import pallas as pl
from jax.experimental.pallas import tpu as pltpu
from jax.experimental.pallas import tpu_sc as plsc

S, H, E, L, Hh = 2048, 768, 8, 3, 384
TB = 256
NB = S // TB
NSTEP = 16
NW = 32
BPW = S // NW


def _dot(a, b):
    return jax.lax.dot_general(a, b, (((1,), (0,)), ((), ())),
                               preferred_element_type=jnp.float32)


def _iota(shape, d):
    return jax.lax.broadcasted_iota(jnp.int32, shape, d).astype(jnp.float32)


def _shift_down(c, k):
    return jnp.concatenate([jnp.zeros((k,) + c.shape[1:], c.dtype), c[:-k]], 0)


def _route_meta(x, W1, b1, W2, b2,
                pos_ref, eid_ref, sb_ref, se_ref, sv_ref, sf_ref):
    h = jnp.maximum(_dot(x, W1) + b1, 0.0)
    scores = _dot(h, W2) + b2
    probs = jax.nn.softmax(scores, axis=-1)
    pmax = jnp.max(probs, axis=1, keepdims=True)
    eidx = jax.lax.broadcasted_iota(jnp.int32, (S, E), 1)
    top = jnp.min(jnp.where(probs == pmax, eidx, E), axis=1, keepdims=True)

    onehot = (top == eidx).astype(jnp.float32)
    c = onehot
    k = 1
    while k < S:
        c = c + _shift_down(c, k)
        k *= 2
    counts = c[S - 1:S, :]
    inc = counts
    k = 1
    while k < E:
        inc = inc + jnp.concatenate(
            [jnp.zeros((1, k), jnp.float32), inc[:, :-k]], 1)
        k *= 2
    offs = inc - counts
    ends = offs + counts

    pos = jnp.sum(onehot * (offs + c - 1.0), axis=1, keepdims=True)
    pos_ref[...] = jnp.reshape(pos.astype(jnp.int32), (16, 128))

    p_iota = _iota((S, E), 0)
    eid = jnp.sum((p_iota >= ends).astype(jnp.float32), axis=1, keepdims=True)
    eid_ref[...] = eid.astype(jnp.int32)

    b_lo = _iota((NB, E), 0) * TB
    flag = ((offs < b_lo + TB) & (ends > b_lo)
            & (counts > 0.0)).astype(jnp.float32)
    inc_e = flag
    k = 1
    while k < E:
        inc_e = inc_e + jnp.concatenate(
            [jnp.zeros((NB, k), jnp.float32), inc_e[:, :-k]], 1)
        k *= 2
    rowtot = inc_e[:, E - 1:E]
    inc_b = rowtot
    k = 1
    while k < NB:
        inc_b = inc_b + _shift_down(inc_b, k)
        k *= 2
    rank = (inc_e - flag) + (inc_b - rowtot)

    t_iota = _iota((NSTEP, NB, E), 0)
    eq = (rank[None] == t_iota) * flag[None]
    b3 = _iota((NSTEP, NB, E), 1)
    e3 = _iota((NSTEP, NB, E), 2)
    sv = jnp.sum(jnp.sum(eq, axis=2, keepdims=True), axis=1)
    sb = jnp.sum(jnp.sum(eq * b3, axis=2, keepdims=True), axis=1)
    se = jnp.sum(jnp.sum(eq * e3, axis=2, keepdims=True), axis=1)
    sb = sb + (1.0 - sv) * (NB - 1)
    se = se + (1.0 - sv) * (E - 1)
    sb_i = sb.astype(jnp.int32)
    prev = jnp.concatenate([jnp.full((1, 1), -1, jnp.int32), sb_i[:-1]], 0)
    sb_ref[...] = sb_i
    se_ref[...] = se.astype(jnp.int32)
    sv_ref[...] = sv.astype(jnp.int32)
    sf_ref[...] = (sb_i != prev).astype(jnp.int32)


_META_SHAPES = [((16, 128), jnp.int32),
                ((S, 1), jnp.int32),
                ((NSTEP, 1), jnp.int32),
                ((NSTEP, 1), jnp.int32),
                ((NSTEP, 1), jnp.int32),
                ((NSTEP, 1), jnp.int32)]


def _router_body(x_ref, W1_ref, b1_ref, W2_ref, b2_ref, *outs):
    _route_meta(x_ref[...], W1_ref[...], b1_ref[...], W2_ref[...],
                b2_ref[...], *outs)


def _run_router(x, W1l, b1l, W2l, b2l):
    return pl.pallas_call(
        _router_body,
        out_shape=[jax.ShapeDtypeStruct(s, d) for s, d in _META_SHAPES],
    )(x, W1l, b1l, W2l, b2l)


def _hw_weight(hw_ref, l):
    w0 = hw_ref[0]
    w1 = hw_ref[1]
    w2 = hw_ref[2]
    m = jnp.maximum(jnp.maximum(w0, w1), w2)
    e0 = jnp.exp(w0 - m)
    e1 = jnp.exp(w1 - m)
    e2 = jnp.exp(w2 - m)
    return (e0, e1, e2)[l] / (e0 + e1 + e2)


def _gmm_body(sb_ref, se_ref, sv_ref, sf_ref, *args, l, last):
    i = 0
    xs_ref = args[i]; i += 1
    if l > 0:
        accin_ref = args[i]; i += 1
    T_ref = args[i]; i += 1
    bb_ref = args[i]; i += 1
    eid_ref = args[i]; i += 1
    a_ref = args[i]; i += 1
    hw_ref = args[i]; i += 1
    if not last:
        W1_ref = args[i]; i += 1
        b1_ref = args[i]; i += 1
        W2_ref = args[i]; i += 1
        b2_ref = args[i]; i += 1
    if not last:
        os_ref = args[i]; i += 1
    acc_ref = args[i]; i += 1
    if not last:
        meta_refs = args[i:i + 6]; i += 6
        os_scr = args[i]; i += 1

    t = pl.program_id(0)
    e = se_ref[t, 0]
    a = a_ref[e]
    valid = sv_ref[t, 0]
    first = sf_ref[t, 0]
    b = sb_ref[t, 0]
    mask = jnp.where(valid > 0,
                     (eid_ref[...] == e).astype(jnp.float32),
                     jnp.zeros_like(eid_ref, jnp.float32))
    Ts = a * T_ref[0]
    contrib = _dot(xs_ref[...] * mask, Ts) + mask * (a * bb_ref[0])
    hw_l = _hw_weight(hw_ref, l)

    if l == 0:
        accc = hw_l * contrib
    else:
        accc = None

    @pl.when(first == 1)
    def _():
        if not last:
            os_ref[...] = contrib
        if l == 0:
            acc_ref[...] = accc
        else:
            acc_ref[...] = accin_ref[...] + hw_l * contrib

    @pl.when(first == 0)
    def _():
        if not last:
            os_ref[...] += contrib
        if l == 0:
            acc_ref[...] += accc
        else:
            acc_ref[...] += hw_l * contrib

    if not last:
        os_scr[pl.ds(b * TB, TB), :] = os_ref[...]

        @pl.when(t == NSTEP - 1)
        def _():
            _route_meta(os_scr[...], W1_ref[...], b1_ref[...], W2_ref[...],
                        b2_ref[...], *meta_refs)


def _run_gmm(l, last, sb, se, sv, sf, xs, accin, Tl, bbl, eid, al, hier_w,
             W1n, b1n, W2n, b2n):
    f32 = jnp.float32
    pfx = lambda f: (lambda t, sb, se, sv, sf: f(t, sb, se, sv, sf))
    in_specs = [pl.BlockSpec((TB, H), lambda t, sb, se, sv, sf: (sb[t, 0], 0))]
    ops = [xs]
    if l > 0:
        in_specs.append(
            pl.BlockSpec((TB, H), lambda t, sb, se, sv, sf: (sb[t, 0], 0)))
        ops.append(accin)
    in_specs += [
        pl.BlockSpec((1, H, H), lambda t, sb, se, sv, sf: (se[t, 0], 0, 0)),
        pl.BlockSpec((1, 1, H), lambda t, sb, se, sv, sf: (se[t, 0], 0, 0)),
        pl.BlockSpec((TB, 1), lambda t, sb, se, sv, sf: (sb[t, 0], 0)),
        pl.BlockSpec(memory_space=pltpu.SMEM),
        pl.BlockSpec(memory_space=pltpu.SMEM),
    ]
    ops += [Tl, bbl.reshape(E, 1, H), eid, al, hier_w]
    if not last:
        in_specs += [pl.BlockSpec((H, Hh), lambda t, sb, se, sv, sf: (0, 0)),
                     pl.BlockSpec((1, Hh), lambda t, sb, se, sv, sf: (0, 0)),
                     pl.BlockSpec((Hh, E), lambda t, sb, se, sv, sf: (0, 0)),
                     pl.BlockSpec((1, E), lambda t, sb, se, sv, sf: (0, 0))]
        ops += [W1n, b1n, W2n, b2n]

    out_specs = []
    out_shape = []
    if not last:
        out_specs.append(
            pl.BlockSpec((TB, H), lambda t, sb, se, sv, sf: (sb[t, 0], 0)))
        out_shape.append(jax.ShapeDtypeStruct((S, H), f32))
    out_specs.append(
        pl.BlockSpec((TB, H), lambda t, sb, se, sv, sf: (sb[t, 0], 0)))
    out_shape.append(jax.ShapeDtypeStruct((S, H), f32))
    scratch = []
    if not last:
        for shp, dt in _META_SHAPES:
            out_specs.append(
                pl.BlockSpec(shp, lambda t, sb, se, sv, sf: tuple(
                    0 for _ in shp)))
            out_shape.append(jax.ShapeDtypeStruct(shp, dt))
        scratch = [pltpu.VMEM((S, H), f32)]

    grid_spec = pltpu.PrefetchScalarGridSpec(
        num_scalar_prefetch=4,
        grid=(NSTEP,),
        in_specs=in_specs,
        out_specs=out_specs,
        scratch_shapes=scratch,
    )
    return pl.pallas_call(
        functools.partial(_gmm_body, l=l, last=last),
        grid_spec=grid_spec,
        out_shape=out_shape,
    )(sb, se, sv, sf, *ops)


def _sc_scatter1(pos2d, x):
    mesh = plsc.VectorSubcoreMesh(core_axis_name="c", subcore_axis_name="s")

    @functools.partial(
        pl.kernel, mesh=mesh,
        out_type=[jax.ShapeDtypeStruct((S, H), jnp.float32),
                  jax.ShapeDtypeStruct((S,), jnp.int32)],
        scratch_types=[pltpu.VMEM((BPW,), jnp.int32),
                       pltpu.VMEM((BPW,), jnp.int32),
                       pltpu.VMEM((BPW, H), jnp.float32),
                       pltpu.SemaphoreType.DMA],
    )
    def k(pos_hbm, x_hbm, out_hbm, inv_hbm, idx_v, ids_v, rows_v, sem):
        wid = lax.axis_index("s") * 2 + lax.axis_index("c")
        pltpu.sync_copy(pos_hbm.at[wid], idx_v)
        pltpu.sync_copy(x_hbm.at[pl.ds(wid * BPW, BPW)], rows_v)
        for j in range(BPW // 16):
            ids_v[pl.ds(j * 16, 16)] = (wid * BPW + j * 16
                                        + lax.iota(jnp.int32, 16))
        cp1 = pltpu.async_copy(rows_v, out_hbm.at[idx_v], sem)
        cp2 = pltpu.async_copy(ids_v, inv_hbm.at[idx_v], sem)
        cp1.wait()
        cp2.wait()

    return k(pos2d, x)


def _sc_scatter2(pos2d, inv2d, x, acc):
    mesh = plsc.VectorSubcoreMesh(core_axis_name="c", subcore_axis_name="s")

    @functools.partial(
        pl.kernel, mesh=mesh,
        out_type=[jax.ShapeDtypeStruct((S, H), jnp.float32),
                  jax.ShapeDtypeStruct((S, H), jnp.float32),
                  jax.ShapeDtypeStruct((S,), jnp.int32)],
        scratch_types=[pltpu.VMEM((BPW,), jnp.int32),
                       pltpu.VMEM((BPW,), jnp.int32),
                       pltpu.VMEM((BPW, H), jnp.float32),
                       pltpu.VMEM((BPW, H), jnp.float32),
                       pltpu.SemaphoreType.DMA],
    )
    def k(pos_hbm, inv_hbm, x_hbm, acc_hbm, xs_hbm, accs_hbm, invn_hbm,
          idx_v, ids_v, rows_v, rows2_v, sem):
        wid = lax.axis_index("s") * 2 + lax.axis_index("c")
        pltpu.sync_copy(pos_hbm.at[wid], idx_v)
        pltpu.sync_copy(inv_hbm.at[wid], ids_v)
        pltpu.sync_copy(x_hbm.at[pl.ds(wid * BPW, BPW)], rows_v)
        pltpu.sync_copy(acc_hbm.at[pl.ds(wid * BPW, BPW)], rows2_v)
        cp1 = pltpu.async_copy(rows_v, xs_hbm.at[idx_v], sem)
        cp2 = pltpu.async_copy(rows2_v, accs_hbm.at[idx_v], sem)
        cp3 = pltpu.async_copy(ids_v, invn_hbm.at[idx_v], sem)
        cp1.wait()
        cp2.wait()
        cp3.wait()

    return k(pos2d, inv2d, x, acc)


def _sc_fin(inv2d, acc):
    mesh = plsc.VectorSubcoreMesh(core_axis_name="c", subcore_axis_name="s")

    @functools.partial(
        pl.kernel, mesh=mesh,
        out_type=jax.ShapeDtypeStruct((S, H), jnp.float32),
        scratch_types=[pltpu.VMEM((BPW,), jnp.int32),
                       pltpu.VMEM((BPW, H), jnp.float32),
                       pltpu.SemaphoreType.DMA],
    )
    def k(inv_hbm, acc_hbm, out_hbm, idx_v, rows_v, sem):
        wid = lax.axis_index("s") * 2 + lax.axis_index("c")
        pltpu.sync_copy(inv_hbm.at[wid], idx_v)
        pltpu.sync_copy(acc_hbm.at[pl.ds(wid * BPW, BPW)], rows_v)
        pltpu.async_copy(rows_v, out_hbm.at[idx_v], sem).wait()

    return k(inv2d, acc)


def kernel(hidden_states, W1, b1, W2, b2, assignments, hier_w, rand_T, rand_b):
    x = hidden_states.reshape(S, H)
    b1r = [b1[l].reshape(1, Hh) for l in range(L)]
    b2r = [b2[l].reshape(1, E) for l in range(L)]

    pos0, eid0, sb, se, sv, sf = _run_router(x, W1[0], b1r[0], W2[0], b2r[0])
    xs, inv0 = _sc_scatter1(pos0.reshape(NW, BPW), x)
    os0, acc0, pos1, eid1, sb1, se1, sv1, sf1 = _run_gmm(
        0, False, sb, se, sv, sf, xs, None, rand_T[0], rand_b[0], eid0,
        assignments[0], hier_w, W1[1], b1r[1], W2[1], b2r[1])
    xs1, accs1, inv1 = _sc_scatter2(pos1.reshape(NW, BPW),
                                    inv0.reshape(NW, BPW), os0, acc0)
    os1, acc1, pos2, eid2, sb2, se2, sv2, sf2 = _run_gmm(
        1, False, sb1, se1, sv1, sf1, xs1, accs1, rand_T[1], rand_b[1], eid1,
        assignments[1], hier_w, W1[2], b1r[2], W2[2], b2r[2])
    xs2, accs2, inv2 = _sc_scatter2(pos2.reshape(NW, BPW),
                                    inv1.reshape(NW, BPW), os1, acc1)
    (acc2,) = _run_gmm(
        2, True, sb2, se2, sv2, sf2, xs2, accs2, rand_T[2], rand_b[2], eid2,
        assignments[2], hier_w, None, None, None, None)
    fin = _sc_fin(inv2.reshape(NW, BPW), acc2)
    return fin.reshape(1, S, H)


# speedup vs baseline: 1.0020x; 1.0020x over previous
import functools

import jax
import jax.numpy as jnp
from jax import lax
from jax.experimental import pallas as pl
from jax.experimental.pallas import tpu as pltpu
from jax.experimental.pallas import tpu_sc as plsc

S, H, E, L, Hh = 2048, 768, 8, 3, 384
TB = 256
NB = S // TB
NSTEP = 16
NW = 32
BPW = S // NW


def _dot(a, b):
    return jax.lax.dot_general(a, b, (((1,), (0,)), ((), ())),
                               preferred_element_type=jnp.float32)


def _iota(shape, d):
    return jax.lax.broadcasted_iota(jnp.int32, shape, d).astype(jnp.float32)


def _shift_down(c, k):
    return jnp.concatenate([jnp.zeros((k,) + c.shape[1:], c.dtype), c[:-k]], 0)


def _route_meta(x, W1, b1, W2, b2,
                pos_ref, eid_ref, sb_ref, se_ref, sv_ref, sf_ref):
    h = jnp.maximum(_dot(x, W1) + b1, 0.0)
    scores = _dot(h, W2) + b2
    probs = jax.nn.softmax(scores, axis=-1)
    pmax = jnp.max(probs, axis=1, keepdims=True)
    eidx = jax.lax.broadcasted_iota(jnp.int32, (S, E), 1)
    top = jnp.min(jnp.where(probs == pmax, eidx, E), axis=1, keepdims=True)

    onehot = (top == eidx).astype(jnp.float32)
    c = onehot
    k = 1
    while k < S:
        c = c + _shift_down(c, k)
        k *= 2
    counts = c[S - 1:S, :]
    inc = counts
    k = 1
    while k < E:
        inc = inc + jnp.concatenate(
            [jnp.zeros((1, k), jnp.float32), inc[:, :-k]], 1)
        k *= 2
    offs = inc - counts
    ends = offs + counts

    pos = jnp.sum(onehot * (offs + c - 1.0), axis=1, keepdims=True)
    pos_ref[...] = jnp.reshape(pos.astype(jnp.int32), (16, 128))

    p_iota = _iota((S, E), 0)
    eid = jnp.sum((p_iota >= ends).astype(jnp.float32), axis=1, keepdims=True)
    eid_ref[...] = eid.astype(jnp.int32)

    b_lo = _iota((NB, E), 0) * TB
    flag = ((offs < b_lo + TB) & (ends > b_lo)
            & (counts > 0.0)).astype(jnp.float32)
    inc_e = flag
    k = 1
    while k < E:
        inc_e = inc_e + jnp.concatenate(
            [jnp.zeros((NB, k), jnp.float32), inc_e[:, :-k]], 1)
        k *= 2
    rowtot = inc_e[:, E - 1:E]
    inc_b = rowtot
    k = 1
    while k < NB:
        inc_b = inc_b + _shift_down(inc_b, k)
        k *= 2
    rank = (inc_e - flag) + (inc_b - rowtot)

    t_iota = _iota((NSTEP, NB, E), 0)
    eq = (rank[None] == t_iota) * flag[None]
    b3 = _iota((NSTEP, NB, E), 1)
    e3 = _iota((NSTEP, NB, E), 2)
    sv = jnp.sum(jnp.sum(eq, axis=2, keepdims=True), axis=1)
    sb = jnp.sum(jnp.sum(eq * b3, axis=2, keepdims=True), axis=1)
    se = jnp.sum(jnp.sum(eq * e3, axis=2, keepdims=True), axis=1)
    sb = sb + (1.0 - sv) * (NB - 1)
    se = se + (1.0 - sv) * (E - 1)
    sb_i = sb.astype(jnp.int32)
    prev = jnp.concatenate([jnp.full((1, 1), -1, jnp.int32), sb_i[:-1]], 0)
    sb_ref[...] = sb_i
    se_ref[...] = se.astype(jnp.int32)
    sv_ref[...] = sv.astype(jnp.int32)
    sf_ref[...] = (sb_i != prev).astype(jnp.int32)


_META_SHAPES = [((16, 128), jnp.int32),
                ((S, 1), jnp.int32),
                ((NSTEP, 1), jnp.int32),
                ((NSTEP, 1), jnp.int32),
                ((NSTEP, 1), jnp.int32),
                ((NSTEP, 1), jnp.int32)]


def _router_body(x_ref, W1_ref, b1_ref, W2_ref, b2_ref, *outs):
    _route_meta(x_ref[...], W1_ref[...], b1_ref[...], W2_ref[...],
                b2_ref[...], *outs)


def _run_router(x, W1l, b1l, W2l, b2l):
    return pl.pallas_call(
        _router_body,
        out_shape=[jax.ShapeDtypeStruct(s, d) for s, d in _META_SHAPES],
    )(x, W1l, b1l, W2l, b2l)


def _hw_weight(hw_ref, l):
    w0 = hw_ref[0]
    w1 = hw_ref[1]
    w2 = hw_ref[2]
    m = jnp.maximum(jnp.maximum(w0, w1), w2)
    e0 = jnp.exp(w0 - m)
    e1 = jnp.exp(w1 - m)
    e2 = jnp.exp(w2 - m)
    return (e0, e1, e2)[l] / (e0 + e1 + e2)


def _gmm_body(sb_ref, se_ref, sv_ref, sf_ref, *args, l, last):
    i = 0
    xs_ref = args[i]; i += 1
    if l > 0:
        accin_ref = args[i]; i += 1
    T_ref = args[i]; i += 1
    bb_ref = args[i]; i += 1
    eid_ref = args[i]; i += 1
    a_ref = args[i]; i += 1
    hw_ref = args[i]; i += 1
    if not last:
        W1_ref = args[i]; i += 1
        b1_ref = args[i]; i += 1
        W2_ref = args[i]; i += 1
        b2_ref = args[i]; i += 1
    if not last:
        os_ref = args[i]; i += 1
    acc_ref = args[i]; i += 1
    if not last:
        meta_refs = args[i:i + 6]; i += 6

    t = pl.program_id(0)
    e = se_ref[t, 0]
    a = a_ref[e]
    valid = sv_ref[t, 0]
    first = sf_ref[t, 0]
    b = sb_ref[t, 0]
    rows = pl.ds(b * TB, TB)
    mask = jnp.where(valid > 0,
                     (eid_ref[...] == e).astype(jnp.float32),
                     jnp.zeros_like(eid_ref, jnp.float32))
    Ts = a * T_ref[0]
    contrib = _dot(xs_ref[...] * mask, Ts) + mask * (a * bb_ref[0])
    hw_l = _hw_weight(hw_ref, l)

    @pl.when(first == 1)
    def _():
        if not last:
            os_ref[rows, :] = contrib
        if l == 0:
            acc_ref[rows, :] = hw_l * contrib
        else:
            acc_ref[rows, :] = accin_ref[...] + hw_l * contrib

    @pl.when(first == 0)
    def _():
        if not last:
            os_ref[rows, :] += contrib
        if l == 0:
            acc_ref[rows, :] += hw_l * contrib
        else:
            acc_ref[rows, :] += hw_l * contrib

    if not last:
        @pl.when(t == NSTEP - 1)
        def _():
            _route_meta(os_ref[...], W1_ref[...], b1_ref[...], W2_ref[...],
                        b2_ref[...], *meta_refs)


def _run_gmm(l, last, sb, se, sv, sf, xs, accin, Tl, bbl, eid, al, hier_w,
             W1n, b1n, W2n, b2n):
    f32 = jnp.float32
    pfx = lambda f: (lambda t, sb, se, sv, sf: f(t, sb, se, sv, sf))
    in_specs = [pl.BlockSpec((TB, H), lambda t, sb, se, sv, sf: (sb[t, 0], 0))]
    ops = [xs]
    if l > 0:
        in_specs.append(
            pl.BlockSpec((TB, H), lambda t, sb, se, sv, sf: (sb[t, 0], 0)))
        ops.append(accin)
    in_specs += [
        pl.BlockSpec((1, H, H), lambda t, sb, se, sv, sf: (se[t, 0], 0, 0)),
        pl.BlockSpec((1, 1, H), lambda t, sb, se, sv, sf: (se[t, 0], 0, 0)),
        pl.BlockSpec((TB, 1), lambda t, sb, se, sv, sf: (sb[t, 0], 0)),
        pl.BlockSpec(memory_space=pltpu.SMEM),
        pl.BlockSpec(memory_space=pltpu.SMEM),
    ]
    ops += [Tl, bbl.reshape(E, 1, H), eid, al, hier_w]
    if not last:
        in_specs += [pl.BlockSpec((H, Hh), lambda t, sb, se, sv, sf: (0, 0)),
                     pl.BlockSpec((1, Hh), lambda t, sb, se, sv, sf: (0, 0)),
                     pl.BlockSpec((Hh, E), lambda t, sb, se, sv, sf: (0, 0)),
                     pl.BlockSpec((1, E), lambda t, sb, se, sv, sf: (0, 0))]
        ops += [W1n, b1n, W2n, b2n]

    out_specs = []
    out_shape = []
    if not last:
        out_specs.append(
            pl.BlockSpec((S, H), lambda t, sb, se, sv, sf: (0, 0)))
        out_shape.append(jax.ShapeDtypeStruct((S, H), f32))
    out_specs.append(
        pl.BlockSpec((S, H), lambda t, sb, se, sv, sf: (0, 0)))
    out_shape.append(jax.ShapeDtypeStruct((S, H), f32))
    if not last:
        for shp, dt in _META_SHAPES:
            out_specs.append(
                pl.BlockSpec(shp, lambda t, sb, se, sv, sf: tuple(
                    0 for _ in shp)))
            out_shape.append(jax.ShapeDtypeStruct(shp, dt))

    grid_spec = pltpu.PrefetchScalarGridSpec(
        num_scalar_prefetch=4,
        grid=(NSTEP,),
        in_specs=in_specs,
        out_specs=out_specs,
    )
    return pl.pallas_call(
        functools.partial(_gmm_body, l=l, last=last),
        grid_spec=grid_spec,
        out_shape=out_shape,
    )(sb, se, sv, sf, *ops)


def _sc_scatter1(pos2d, x):
    mesh = plsc.VectorSubcoreMesh(core_axis_name="c", subcore_axis_name="s")

    @functools.partial(
        pl.kernel, mesh=mesh,
        out_type=[jax.ShapeDtypeStruct((S, H), jnp.float32),
                  jax.ShapeDtypeStruct((S,), jnp.int32)],
        scratch_types=[pltpu.VMEM((BPW,), jnp.int32),
                       pltpu.VMEM((BPW,), jnp.int32),
                       pltpu.VMEM((BPW, H), jnp.float32),
                       pltpu.SemaphoreType.DMA],
    )
    def k(pos_hbm, x_hbm, out_hbm, inv_hbm, idx_v, ids_v, rows_v, sem):
        wid = lax.axis_index("s") * 2 + lax.axis_index("c")
        pltpu.sync_copy(pos_hbm.at[wid], idx_v)
        pltpu.sync_copy(x_hbm.at[pl.ds(wid * BPW, BPW)], rows_v)
        for j in range(BPW // 16):
            ids_v[pl.ds(j * 16, 16)] = (wid * BPW + j * 16
                                        + lax.iota(jnp.int32, 16))
        cp1 = pltpu.async_copy(rows_v, out_hbm.at[idx_v], sem)
        cp2 = pltpu.async_copy(ids_v, inv_hbm.at[idx_v], sem)
        cp1.wait()
        cp2.wait()

    return k(pos2d, x)


def _sc_scatter2(pos2d, inv2d, x, acc):
    mesh = plsc.VectorSubcoreMesh(core_axis_name="c", subcore_axis_name="s")

    @functools.partial(
        pl.kernel, mesh=mesh,
        out_type=[jax.ShapeDtypeStruct((S, H), jnp.float32),
                  jax.ShapeDtypeStruct((S, H), jnp.float32),
                  jax.ShapeDtypeStruct((S,), jnp.int32)],
        scratch_types=[pltpu.VMEM((BPW,), jnp.int32),
                       pltpu.VMEM((BPW,), jnp.int32),
                       pltpu.VMEM((BPW, H), jnp.float32),
                       pltpu.VMEM((BPW, H), jnp.float32),
                       pltpu.SemaphoreType.DMA],
    )
    def k(pos_hbm, inv_hbm, x_hbm, acc_hbm, xs_hbm, accs_hbm, invn_hbm,
          idx_v, ids_v, rows_v, rows2_v, sem):
        wid = lax.axis_index("s") * 2 + lax.axis_index("c")
        pltpu.sync_copy(pos_hbm.at[wid], idx_v)
        pltpu.sync_copy(inv_hbm.at[wid], ids_v)
        pltpu.sync_copy(x_hbm.at[pl.ds(wid * BPW, BPW)], rows_v)
        pltpu.sync_copy(acc_hbm.at[pl.ds(wid * BPW, BPW)], rows2_v)
        cp1 = pltpu.async_copy(rows_v, xs_hbm.at[idx_v], sem)
        cp2 = pltpu.async_copy(rows2_v, accs_hbm.at[idx_v], sem)
        cp3 = pltpu.async_copy(ids_v, invn_hbm.at[idx_v], sem)
        cp1.wait()
        cp2.wait()
        cp3.wait()

    return k(pos2d, inv2d, x, acc)


def _sc_fin(inv2d, acc):
    mesh = plsc.VectorSubcoreMesh(core_axis_name="c", subcore_axis_name="s")

    @functools.partial(
        pl.kernel, mesh=mesh,
        out_type=jax.ShapeDtypeStruct((S, H), jnp.float32),
        scratch_types=[pltpu.VMEM((BPW,), jnp.int32),
                       pltpu.VMEM((BPW, H), jnp.float32),
                       pltpu.SemaphoreType.DMA],
    )
    def k(inv_hbm, acc_hbm, out_hbm, idx_v, rows_v, sem):
        wid = lax.axis_index("s") * 2 + lax.axis_index("c")
        pltpu.sync_copy(inv_hbm.at[wid], idx_v)
        pltpu.sync_copy(acc_hbm.at[pl.ds(wid * BPW, BPW)], rows_v)
        pltpu.async_copy(rows_v, out_hbm.at[idx_v], sem).wait()

    return k(inv2d, acc)


def kernel(hidden_states, W1, b1, W2, b2, assignments, hier_w, rand_T, rand_b):
    x = hidden_states.reshape(S, H)
    b1r = [b1[l].reshape(1, Hh) for l in range(L)]
    b2r = [b2[l].reshape(1, E) for l in range(L)]

    pos0, eid0, sb, se, sv, sf = _run_router(x, W1[0], b1r[0], W2[0], b2r[0])
    xs, inv0 = _sc_scatter1(pos0.reshape(NW, BPW), x)
    os0, acc0, pos1, eid1, sb1, se1, sv1, sf1 = _run_gmm(
        0, False, sb, se, sv, sf, xs, None, rand_T[0], rand_b[0], eid0,
        assignments[0], hier_w, W1[1], b1r[1], W2[1], b2r[1])
    xs1, accs1, inv1 = _sc_scatter2(pos1.reshape(NW, BPW),
                                    inv0.reshape(NW, BPW), os0, acc0)
    os1, acc1, pos2, eid2, sb2, se2, sv2, sf2 = _run_gmm(
        1, False, sb1, se1, sv1, sf1, xs1, accs1, rand_T[1], rand_b[1], eid1,
        assignments[1], hier_w, W1[2], b1r[2], W2[2], b2r[2])
    xs2, accs2, inv2 = _sc_scatter2(pos2.reshape(NW, BPW),
                                    inv1.reshape(NW, BPW), os1, acc1)
    (acc2,) = _run_gmm(
        2, True, sb2, se2, sv2, sf2, xs2, accs2, rand_T[2], rand_b[2], eid2,
        assignments[2], hier_w, None, None, None, None)
    fin = _sc_fin(inv2.reshape(NW, BPW), acc2)
    return fin.reshape(1, S, H)
